# speedup vs baseline: 1.5140x; 1.5140x over previous
import jax
import jax.numpy as jnp
from jax.experimental import pallas as pl
from jax.experimental.pallas import tpu as pltpu

BUCKET_SIZE = 32
DIM = 128


def _body(q_ref, k_ref, o_ref):
    n, d = q_ref.shape[1], q_ref.shape[2]
    buckets = n // BUCKET_SIZE
    qb = q_ref[0].reshape(buckets, BUCKET_SIZE, d)
    kb = k_ref[0].reshape(buckets, BUCKET_SIZE, d)
    sq = jnp.sum(qb, axis=1) * (1.0 / BUCKET_SIZE)
    sk = jnp.sum(kb, axis=1) * (1.0 / BUCKET_SIZE)
    r = jax.lax.dot_general(
        sq, sk, (((1,), (1,)), ((), ())),
        preferred_element_type=jnp.float32) * (DIM ** -0.5)
    m = jnp.max(r, axis=-1, keepdims=True)
    e = jnp.exp(r - m)
    o_ref[0] = e / jnp.sum(e, axis=-1, keepdims=True)


def kernel(q, k):
    bh, n, d = q.shape
    buckets = n // BUCKET_SIZE
    return pl.pallas_call(
        _body,
        grid=(bh,),
        in_specs=[
            pl.BlockSpec((1, n, d), lambda i: (i, 0, 0)),
            pl.BlockSpec((1, n, d), lambda i: (i, 0, 0)),
        ],
        out_specs=pl.BlockSpec((1, buckets, buckets), lambda i: (i, 0, 0)),
        out_shape=jax.ShapeDtypeStruct((bh, buckets, buckets), jnp.float32),
    )(q, k)
